# Initial kernel scaffold; baseline (speedup 1.0000x reference)
import jax
import jax.numpy as jnp
from jax.experimental import pallas as pl


def kernel(x_in, w1_0, b1_0, w1_1, b1_1, w1_2, b1_2, w2_0, b2_0, w2_1, b2_1, w2_2, b2_2, w3_0, b3_0, w3_1, b3_1, w3_2, b3_2):
    raise NotImplementedError("write your pallas kernel here")


# speedup vs baseline: 9.3768x; 9.3768x over previous
import functools

import jax
import jax.numpy as jnp
from jax import lax
from jax.experimental import pallas as pl
from jax.experimental.pallas import tpu as pltpu
from jax.experimental.pallas import tpu_sc as plsc

H = 224
W = 256
C1, C2, C3 = 16, 12, 8
PAD2, PAD3 = 2, 4
C12 = C1 * C2
NPIX = H * W
NUM2 = H * C1
NUM3 = H * C12

_F32 = jnp.float32
_I32 = jnp.int32


def _lrelu(x):
    return jnp.where(x >= 0, x, 0.01 * x)


def _mm(a, b):
    return lax.dot_general(
        a, b, (((1,), (0,)), ((), ())),
        preferred_element_type=_F32, precision=lax.Precision.DEFAULT)


def _first_argmax_rows(a):
    m = jnp.max(a, axis=0, keepdims=True)
    io = lax.broadcasted_iota(_I32, a.shape, 0)
    return jnp.min(jnp.where(a == m, io, jnp.int32(10 ** 6)), axis=0,
                   keepdims=True)


def _stage12_body(x_ref, w10, b10, w11, b11, w12, b12,
                  w20, b20, w21, b21, w22, b22, i12_ref, i12l_ref):
    h = pl.program_id(0)
    xa = x_ref[0, 0:32, :]
    xb = x_ref[0, 32:64, :]

    a = _lrelu(_mm(w10[0], xa) + b10[0])
    a = _lrelu(_mm(w11[0], a) + b11[0])
    a = _mm(w12[0], a) + b12[0]
    ind1 = _first_argmax_rows(a)

    def cond_layer(x, w_ref, b_ref, cout):
        wr = w_ref[...].reshape(16 * cout, 32)
        y = _mm(wr, x) + b_ref[...].reshape(16 * cout, 1)
        sel = jnp.zeros((cout, W), _F32)
        for e in range(16):
            sel = jnp.where(ind1 == e, y[e * cout:(e + 1) * cout, :], sel)
        return sel

    t = _lrelu(cond_layer(xb, w20, b20, 32))
    t = _lrelu(cond_layer(t, w21, b21, 32))
    t = cond_layer(t, w22, b22, C1)
    ind2 = _first_argmax_rows(t)

    i12 = ind1 * C2 + (ind2 - PAD2)
    i12l = jnp.clip(i12, 0, C12 - 1) + C12 * h
    i12_ref[0] = i12
    i12l_ref[0] = i12l


def _stage12(xr, w1_0, b1_0, w1_1, b1_1, w1_2, b1_2,
             w2t0, b2r0, w2t1, b2r1, w2t2, b2r2):
    grid = (H,)
    line3 = lambda shape: pl.BlockSpec(shape, lambda h: (h, 0, 0))
    out_sd = jax.ShapeDtypeStruct((H, 1, W), _I32)
    return pl.pallas_call(
        _stage12_body,
        grid=grid,
        in_specs=[
            pl.BlockSpec((1, 96, W), lambda h: (h, 0, 0)),
            line3((1, 32, 32)), line3((1, 32, 1)),
            line3((1, 32, 32)), line3((1, 32, 1)),
            line3((1, 16, 32)), line3((1, 16, 1)),
            line3((16, 32, 32)), line3((16, 32, 1)),
            line3((16, 32, 32)), line3((16, 32, 1)),
            line3((16, 16, 32)), line3((16, 16, 1)),
        ],
        out_specs=[line3((1, 1, W)), line3((1, 1, W))],
        out_shape=[out_sd, out_sd],
    )(xr, w1_0, b1_0, w1_1, b1_1, w1_2, b1_2,
      w2t0, b2r0, w2t1, b2r1, w2t2, b2r2)


_NW = 32
_PW = NPIX // _NW
_GP = 16
_NG = _PW // _GP


def _sc_mesh():
    return plsc.VectorSubcoreMesh(core_axis_name="c", subcore_axis_name="s",
                                  num_cores=2, num_subcores=16)


def _stage3_body(xc, idxh, i12h, w30, w31, w32, bcat, outh,
                 idx_v, xbuf, h1buf, h2buf, w0b, w1b, w2b, bb,
                 i12b, resb):
    cid = lax.axis_index("c")
    sid = lax.axis_index("s")
    wid = sid * 2 + cid
    lanes = lax.iota(_I32, 16)

    def group(g, carry):
        base = wid * _PW + g * _GP
        pltpu.sync_copy(idxh.at[pl.ds(base, _GP)], idx_v)
        pltpu.sync_copy(i12h.at[pl.ds(base, _GP)], i12b)
        pltpu.sync_copy(xc.at[pl.ds(base, _GP)], xbuf)
        pltpu.sync_copy(w30.at[idx_v], w0b)
        pltpu.sync_copy(w31.at[idx_v], w1b)
        pltpu.sync_copy(w32.at[idx_v], w2b)
        pltpu.sync_copy(bcat.at[idx_v], bb)

        def layer0(o, _):
            o_idx = jnp.full((16,), 0, _I32) + o
            acc = plsc.load_gather(bb, [lanes, o_idx])
            for i in range(32):
                xi = plsc.load_gather(xbuf, [lanes, jnp.full((16,), i, _I32)])
                acc = acc + xi * plsc.load_gather(w0b, [lanes, o_idx + i * 32])
            acc = jnp.where(acc >= 0, acc, 0.01 * acc)
            plsc.store_scatter(h1buf, [o_idx, lanes], acc)
            return 0

        def layer1(o, _):
            o_idx = jnp.full((16,), 0, _I32) + o
            acc = plsc.load_gather(bb, [lanes, o_idx + 32])
            for i in range(32):
                acc = acc + h1buf[i, :] * plsc.load_gather(
                    w1b, [lanes, o_idx + i * 32])
            acc = jnp.where(acc >= 0, acc, 0.01 * acc)
            plsc.store_scatter(h2buf, [o_idx, lanes], acc)
            return 0

        lax.fori_loop(0, 32, layer0, 0)
        lax.fori_loop(0, 32, layer1, 0)

        def layer2(o, mc):
            m, ind = mc
            o_idx = jnp.full((16,), 0, _I32) + o
            acc = plsc.load_gather(bb, [lanes, o_idx + 64])
            for i in range(32):
                acc = acc + h2buf[i, :] * plsc.load_gather(
                    w2b, [lanes, o_idx + i * C1])
            upd = acc > m
            return jnp.where(upd, acc, m), jnp.where(upd, o_idx, ind)

        minit = jnp.full((16,), -jnp.inf, _F32)
        iinit = jnp.full((16,), 0, _I32)
        _, ind3 = lax.fori_loop(0, C1, layer2, (minit, iinit))

        i12v = i12b[...]
        res = jnp.clip(i12v * C3 + (ind3 - PAD3), 0, C12 * C3 - 1)
        resb[...] = res
        pltpu.sync_copy(resb, outh.at[pl.ds(base, _GP)])
        return carry

    lax.fori_loop(0, _NG, group, 0)


def _stage3(xc, i12l, i12, w30r, w31r, w32r, bcat):
    fn = pl.kernel(
        _stage3_body,
        out_type=jax.ShapeDtypeStruct((NPIX,), _I32),
        mesh=_sc_mesh(),
        compiler_params=pltpu.CompilerParams(needs_layout_passes=False),
        scratch_types=[
            pltpu.VMEM((_GP,), _I32),
            pltpu.VMEM((_GP, 32), _F32),
            pltpu.VMEM((32, _GP), _F32),
            pltpu.VMEM((32, _GP), _F32),
            pltpu.VMEM((_GP, 1024), _F32),
            pltpu.VMEM((_GP, 1024), _F32),
            pltpu.VMEM((_GP, 512), _F32),
            pltpu.VMEM((_GP, 128), _F32),
            pltpu.VMEM((_GP,), _I32),
            pltpu.VMEM((_GP,), _I32),
        ],
    )
    return fn(xc, i12l, i12, w30r, w31r, w32r, bcat)


def kernel(x_in, w1_0, b1_0, w1_1, b1_1, w1_2, b1_2,
           w2_0, b2_0, w2_1, b2_1, w2_2, b2_2,
           w3_0, b3_0, w3_1, b3_1, w3_2, b3_2):
    w2t0 = w2_0.transpose(0, 2, 1)
    w2t1 = w2_1.transpose(0, 2, 1)
    w2t2 = w2_2.transpose(0, 2, 1)

    xr = x_in[0].transpose(1, 0, 2)
    i12_3d, i12l_3d = _stage12(
        xr, w1_0, b1_0[:, :, None], w1_1, b1_1[:, :, None],
        w1_2, b1_2[:, :, None],
        w2t0, b2_0[:, :, None], w2t1, b2_1[:, :, None],
        w2t2, b2_2[:, :, None])
    i12 = i12_3d.reshape(NPIX)
    i12l = i12l_3d.reshape(NPIX)

    xc = x_in[0, 64:96].reshape(32, NPIX).T
    w30r = w3_0.reshape(NUM3, 32 * 32)
    w31r = w3_1.reshape(NUM3, 32 * 32)
    w32r = w3_2.reshape(NUM3, 32 * C1)
    bcat = jnp.concatenate(
        [b3_0, b3_1, b3_2, jnp.zeros((NUM3, 128 - 64 - C1), _F32)], axis=1)
    out = _stage3(xc, i12l, i12, w30r, w31r, w32r, bcat)
    return out.reshape(1, 1, H, W)


# speedup vs baseline: 11.0082x; 1.1740x over previous
import functools

import jax
import jax.numpy as jnp
from jax import lax
from jax.experimental import pallas as pl
from jax.experimental.pallas import tpu as pltpu
from jax.experimental.pallas import tpu_sc as plsc

H = 224
W = 256
C1, C2, C3 = 16, 12, 8
PAD2, PAD3 = 2, 4
C12 = C1 * C2
NPIX = H * W
NUM2 = H * C1
NUM3 = H * C12

_F32 = jnp.float32
_I32 = jnp.int32


def _lrelu(x):
    return jnp.where(x >= 0, x, 0.01 * x)


def _mm(a, b):
    return lax.dot_general(
        a, b, (((1,), (0,)), ((), ())),
        preferred_element_type=_F32, precision=lax.Precision.DEFAULT)


def _first_argmax_rows(a):
    m = jnp.max(a, axis=0, keepdims=True)
    io = lax.broadcasted_iota(_I32, a.shape, 0)
    return jnp.min(jnp.where(a == m, io, jnp.int32(10 ** 6)), axis=0,
                   keepdims=True)


def _stage12_body(x_ref, w10, b10, w11, b11, w12, b12,
                  w20, b20, w21, b21, w22, b22, i12_ref, i12l_ref):
    h = pl.program_id(0)
    xa = x_ref[0, 0:32, :]
    xb = x_ref[0, 32:64, :]

    a = _lrelu(_mm(w10[0], xa) + b10[0])
    a = _lrelu(_mm(w11[0], a) + b11[0])
    a = _mm(w12[0], a) + b12[0]
    ind1 = _first_argmax_rows(a)

    def cond_layer(x, w_ref, b_ref, cout):
        wr = w_ref[...].reshape(16 * cout, 32)
        y = _mm(wr, x) + b_ref[...].reshape(16 * cout, 1)
        sel = jnp.zeros((cout, W), _F32)
        for e in range(16):
            sel = jnp.where(ind1 == e, y[e * cout:(e + 1) * cout, :], sel)
        return sel

    t = _lrelu(cond_layer(xb, w20, b20, 32))
    t = _lrelu(cond_layer(t, w21, b21, 32))
    t = cond_layer(t, w22, b22, C1)
    ind2 = _first_argmax_rows(t)

    i12 = ind1 * C2 + (ind2 - PAD2)
    i12l = jnp.clip(i12, 0, C12 - 1) + C12 * h
    i12_ref[0] = i12
    i12l_ref[0] = i12l


def _stage12(xr, w1_0, b1_0, w1_1, b1_1, w1_2, b1_2,
             w2t0, b2r0, w2t1, b2r1, w2t2, b2r2):
    grid = (H,)
    line3 = lambda shape: pl.BlockSpec(shape, lambda h: (h, 0, 0))
    out_sd = jax.ShapeDtypeStruct((H, 1, W), _I32)
    return pl.pallas_call(
        _stage12_body,
        grid=grid,
        in_specs=[
            pl.BlockSpec((1, 96, W), lambda h: (h, 0, 0)),
            line3((1, 32, 32)), line3((1, 32, 1)),
            line3((1, 32, 32)), line3((1, 32, 1)),
            line3((1, 16, 32)), line3((1, 16, 1)),
            line3((16, 32, 32)), line3((16, 32, 1)),
            line3((16, 32, 32)), line3((16, 32, 1)),
            line3((16, 16, 32)), line3((16, 16, 1)),
        ],
        out_specs=[line3((1, 1, W)), line3((1, 1, W))],
        out_shape=[out_sd, out_sd],
    )(xr, w1_0, b1_0, w1_1, b1_1, w1_2, b1_2,
      w2t0, b2r0, w2t1, b2r1, w2t2, b2r2)


_NW = 32
_PW = NPIX // _NW
_GP = 16
_NG = _PW // _GP


def _sc_mesh():
    return plsc.VectorSubcoreMesh(core_axis_name="c", subcore_axis_name="s",
                                  num_cores=2, num_subcores=16)


def _stage3_body(xc, idxh, i12h, w30, w31, w32, bcat, outh,
                 idx_all, i12_all, res_all, h1buf, h2buf,
                 xb_0, xb_1, w0_0, w0_1, w1_0b, w1_1b, w2_0b, w2_1b,
                 bb_0, bb_1, sem0, sem1):
    cid = lax.axis_index("c")
    sid = lax.axis_index("s")
    wid = sid * 2 + cid
    lanes = lax.iota(_I32, 16)
    xbufs, w0bufs, w1bufs = (xb_0, xb_1), (w0_0, w0_1), (w1_0b, w1_1b)
    w2bufs, bbufs, sems = (w2_0b, w2_1b), (bb_0, bb_1), (sem0, sem1)

    pltpu.sync_copy(idxh.at[pl.ds(wid * _PW, _PW)], idx_all)
    pltpu.sync_copy(i12h.at[pl.ds(wid * _PW, _PW)], i12_all)

    def copies(g, b):
        idxs = idx_all.at[pl.ds(g * _GP, _GP)]
        return (
            (xc.at[pl.ds(wid * _PW + g * _GP, _GP)], xbufs[b]),
            (w30.at[idxs], w0bufs[b]),
            (w31.at[idxs], w1bufs[b]),
            (w32.at[idxs], w2bufs[b]),
            (bcat.at[idxs], bbufs[b]),
        )

    def issue(g, b):
        for src, dst in copies(g, b):
            pltpu.async_copy(src, dst, sems[b])

    def drain(g, b):
        for src, dst in copies(g, b):
            pltpu.make_async_copy(src, dst, sems[b]).wait()

    def compute(g, b):
        xbuf, w0b, w1b = xbufs[b], w0bufs[b], w1bufs[b]
        w2b, bb = w2bufs[b], bbufs[b]

        def layer0(o, _):
            o_idx = jnp.full((16,), 0, _I32) + o
            acc = plsc.load_gather(bb, [lanes, o_idx])
            for i in range(32):
                xi = plsc.load_gather(xbuf, [lanes, jnp.full((16,), i, _I32)])
                acc = acc + xi * plsc.load_gather(w0b, [lanes, o_idx + i * 32])
            acc = jnp.where(acc >= 0, acc, 0.01 * acc)
            plsc.store_scatter(h1buf, [o_idx, lanes], acc)
            return 0

        def layer1(o, _):
            o_idx = jnp.full((16,), 0, _I32) + o
            acc = plsc.load_gather(bb, [lanes, o_idx + 32])
            for i in range(32):
                acc = acc + h1buf[i, :] * plsc.load_gather(
                    w1b, [lanes, o_idx + i * 32])
            acc = jnp.where(acc >= 0, acc, 0.01 * acc)
            plsc.store_scatter(h2buf, [o_idx, lanes], acc)
            return 0

        lax.fori_loop(0, 32, layer0, 0)
        lax.fori_loop(0, 32, layer1, 0)

        def layer2(o, mc):
            m, ind = mc
            o_idx = jnp.full((16,), 0, _I32) + o
            acc = plsc.load_gather(bb, [lanes, o_idx + 64])
            for i in range(32):
                acc = acc + h2buf[i, :] * plsc.load_gather(
                    w2b, [lanes, o_idx + i * C1])
            upd = acc > m
            return jnp.where(upd, acc, m), jnp.where(upd, o_idx, ind)

        minit = jnp.full((16,), -jnp.inf, _F32)
        iinit = jnp.full((16,), 0, _I32)
        _, ind3 = lax.fori_loop(0, C1, layer2, (minit, iinit))

        i12v = i12_all[pl.ds(g * _GP, _GP)]
        res = jnp.clip(i12v * C3 + (ind3 - PAD3), 0, C12 * C3 - 1)
        plsc.store_scatter(res_all, [g * _GP + lanes], res)

    issue(0, 0)
    issue(1, 1)

    def body(gp, carry):
        for b in (0, 1):
            g = gp * 2 + b
            drain(g, b)
            compute(g, b)

            @pl.when(g + 2 < _NG)
            def _():
                issue(g + 2, b)
        return carry

    lax.fori_loop(0, _NG // 2, body, 0)
    pltpu.sync_copy(res_all, outh.at[pl.ds(wid * _PW, _PW)])


def _stage3(xc, i12l, i12, w30r, w31r, w32r, bcat):
    fn = pl.kernel(
        _stage3_body,
        out_type=jax.ShapeDtypeStruct((NPIX,), _I32),
        mesh=_sc_mesh(),
        compiler_params=pltpu.CompilerParams(needs_layout_passes=False),
        scratch_types=[
            pltpu.VMEM((_PW,), _I32),
            pltpu.VMEM((_PW,), _I32),
            pltpu.VMEM((_PW,), _I32),
            pltpu.VMEM((32, _GP), _F32),
            pltpu.VMEM((32, _GP), _F32),
            pltpu.VMEM((_GP, 32), _F32),
            pltpu.VMEM((_GP, 32), _F32),
            pltpu.VMEM((_GP, 1024), _F32),
            pltpu.VMEM((_GP, 1024), _F32),
            pltpu.VMEM((_GP, 1024), _F32),
            pltpu.VMEM((_GP, 1024), _F32),
            pltpu.VMEM((_GP, 512), _F32),
            pltpu.VMEM((_GP, 512), _F32),
            pltpu.VMEM((_GP, 128), _F32),
            pltpu.VMEM((_GP, 128), _F32),
            pltpu.SemaphoreType.DMA,
            pltpu.SemaphoreType.DMA,
        ],
    )
    return fn(xc, i12l, i12, w30r, w31r, w32r, bcat)


def kernel(x_in, w1_0, b1_0, w1_1, b1_1, w1_2, b1_2,
           w2_0, b2_0, w2_1, b2_1, w2_2, b2_2,
           w3_0, b3_0, w3_1, b3_1, w3_2, b3_2):
    w2t0 = w2_0.transpose(0, 2, 1)
    w2t1 = w2_1.transpose(0, 2, 1)
    w2t2 = w2_2.transpose(0, 2, 1)

    xr = x_in[0].transpose(1, 0, 2)
    i12_3d, i12l_3d = _stage12(
        xr, w1_0, b1_0[:, :, None], w1_1, b1_1[:, :, None],
        w1_2, b1_2[:, :, None],
        w2t0, b2_0[:, :, None], w2t1, b2_1[:, :, None],
        w2t2, b2_2[:, :, None])
    i12 = i12_3d.reshape(NPIX)
    i12l = i12l_3d.reshape(NPIX)

    xc = x_in[0, 64:96].reshape(32, NPIX).T
    w30r = w3_0.reshape(NUM3, 32 * 32)
    w31r = w3_1.reshape(NUM3, 32 * 32)
    w32r = w3_2.reshape(NUM3, 32 * C1)
    bcat = jnp.concatenate(
        [b3_0, b3_1, b3_2, jnp.zeros((NUM3, 128 - 64 - C1), _F32)], axis=1)
    out = _stage3(xc, i12l, i12, w30r, w31r, w32r, bcat)
    return out.reshape(1, 1, H, W)


# speedup vs baseline: 39.4681x; 3.5853x over previous
import functools

import jax
import jax.numpy as jnp
from jax import lax
from jax.experimental import pallas as pl
from jax.experimental.pallas import tpu as pltpu
from jax.experimental.pallas import tpu_sc as plsc

H = 224
W = 256
C1, C2, C3 = 16, 12, 8
PAD2, PAD3 = 2, 4
C12 = C1 * C2
NPIX = H * W
NUM2 = H * C1
NUM3 = H * C12

_F32 = jnp.float32
_I32 = jnp.int32


def _lrelu(x):
    return jnp.where(x >= 0, x, 0.01 * x)


def _mm(a, b):
    return lax.dot_general(
        a, b, (((1,), (0,)), ((), ())),
        preferred_element_type=_F32, precision=lax.Precision.DEFAULT)


def _first_argmax_rows(a):
    m = jnp.max(a, axis=0, keepdims=True)
    io = lax.broadcasted_iota(_I32, a.shape, 0)
    return jnp.min(jnp.where(a == m, io, jnp.int32(10 ** 6)), axis=0,
                   keepdims=True)


def _stage12_body(x_ref, w10, b10, w11, b11, w12, b12,
                  w20, b20, w21, b21, w22, b22, i12_ref, i12l_ref):
    h = pl.program_id(0)
    xa = x_ref[0, 0:32, :]
    xb = x_ref[0, 32:64, :]

    a = _lrelu(_mm(w10[0], xa) + b10[0])
    a = _lrelu(_mm(w11[0], a) + b11[0])
    a = _mm(w12[0], a) + b12[0]
    ind1 = _first_argmax_rows(a)

    def cond_layer(x, w_ref, b_ref, cout):
        wr = w_ref[...].reshape(16 * cout, 32)
        y = _mm(wr, x) + b_ref[...].reshape(16 * cout, 1)
        sel = jnp.zeros((cout, W), _F32)
        for e in range(16):
            sel = jnp.where(ind1 == e, y[e * cout:(e + 1) * cout, :], sel)
        return sel

    t = _lrelu(cond_layer(xb, w20, b20, 32))
    t = _lrelu(cond_layer(t, w21, b21, 32))
    t = cond_layer(t, w22, b22, C1)
    ind2 = _first_argmax_rows(t)

    i12 = ind1 * C2 + (ind2 - PAD2)
    i12l = jnp.clip(i12, 0, C12 - 1) + C12 * h
    i12_ref[0] = i12
    i12l_ref[0] = i12l


def _stage12(xr, w1_0, b1_0, w1_1, b1_1, w1_2, b1_2,
             w2t0, b2r0, w2t1, b2r1, w2t2, b2r2):
    grid = (H,)
    line3 = lambda shape: pl.BlockSpec(shape, lambda h: (h, 0, 0))
    out_sd = jax.ShapeDtypeStruct((H, 1, W), _I32)
    return pl.pallas_call(
        _stage12_body,
        grid=grid,
        in_specs=[
            pl.BlockSpec((1, 96, W), lambda h: (h, 0, 0)),
            line3((1, 32, 32)), line3((1, 32, 1)),
            line3((1, 32, 32)), line3((1, 32, 1)),
            line3((1, 16, 32)), line3((1, 16, 1)),
            line3((16, 32, 32)), line3((16, 32, 1)),
            line3((16, 32, 32)), line3((16, 32, 1)),
            line3((16, 16, 32)), line3((16, 16, 1)),
        ],
        out_specs=[line3((1, 1, W)), line3((1, 1, W))],
        out_shape=[out_sd, out_sd],
    )(xr, w1_0, b1_0, w1_1, b1_1, w1_2, b1_2,
      w2t0, b2r0, w2t1, b2r1, w2t2, b2r2)


_NW = 32
_PW = NPIX // _NW
_GP = 16
_NG = _PW // _GP


def _sc_mesh():
    return plsc.VectorSubcoreMesh(core_axis_name="c", subcore_axis_name="s",
                                  num_cores=2, num_subcores=16)


def _stage3_body(xc, idxh, i12h, w30, w31, w32, bcat, outh,
                 idx_all, i12_all, res_all, h1buf, h2buf,
                 xb_0, xb_1, w0_0, w0_1, w1_0b, w1_1b, w2_0b, w2_1b,
                 bb_0, bb_1, sem0, sem1):
    cid = lax.axis_index("c")
    sid = lax.axis_index("s")
    wid = sid * 2 + cid
    lanes = lax.iota(_I32, 16)
    xbufs, w0bufs, w1bufs = (xb_0, xb_1), (w0_0, w0_1), (w1_0b, w1_1b)
    w2bufs, bbufs, sems = (w2_0b, w2_1b), (bb_0, bb_1), (sem0, sem1)

    pltpu.sync_copy(idxh.at[pl.ds(wid * _PW, _PW)], idx_all)
    pltpu.sync_copy(i12h.at[pl.ds(wid * _PW, _PW)], i12_all)

    def copies(g, b):
        idxs = idx_all.at[pl.ds(g * _GP, _GP)]
        return (
            (xc.at[pl.ds(wid * _PW + g * _GP, _GP)], xbufs[b]),
            (w30.at[idxs], w0bufs[b]),
            (w31.at[idxs], w1bufs[b]),
            (w32.at[idxs], w2bufs[b]),
            (bcat.at[idxs], bbufs[b]),
        )

    def issue(g, b):
        for src, dst in copies(g, b):
            pltpu.async_copy(src, dst, sems[b])

    def drain(g, b):
        for src, dst in copies(g, b):
            pltpu.make_async_copy(src, dst, sems[b]).wait()

    def compute(g, b):
        xbuf, w0b, w1b = xbufs[b], w0bufs[b], w1bufs[b]
        w2b, bb = w2bufs[b], bbufs[b]

        def bcast(lo, hi, i):
            src = lo if i < 16 else hi
            idx = jnp.full((16, 1), i % 16, _I32)
            return lax.gather(
                src, idx,
                lax.GatherDimensionNumbers(
                    offset_dims=(), collapsed_slice_dims=(0,),
                    start_index_map=(0,)),
                (1,), mode=lax.GatherScatterMode.PROMISE_IN_BOUNDS)

        def pix(p, res):
            x_lo = xbuf[p, pl.ds(0, 16)]
            x_hi = xbuf[p, pl.ds(16, 16)]
            a_lo = bb[p, pl.ds(0, 16)]
            a_hi = bb[p, pl.ds(16, 16)]
            for i in range(32):
                xs = bcast(x_lo, x_hi, i)
                a_lo = a_lo + xs * w0b[p, pl.ds(i * 32, 16)]
                a_hi = a_hi + xs * w0b[p, pl.ds(i * 32 + 16, 16)]
            a_lo = jnp.where(a_lo >= 0, a_lo, 0.01 * a_lo)
            a_hi = jnp.where(a_hi >= 0, a_hi, 0.01 * a_hi)

            c_lo = bb[p, pl.ds(32, 16)]
            c_hi = bb[p, pl.ds(48, 16)]
            for i in range(32):
                hs = bcast(a_lo, a_hi, i)
                c_lo = c_lo + hs * w1b[p, pl.ds(i * 32, 16)]
                c_hi = c_hi + hs * w1b[p, pl.ds(i * 32 + 16, 16)]
            c_lo = jnp.where(c_lo >= 0, c_lo, 0.01 * c_lo)
            c_hi = jnp.where(c_hi >= 0, c_hi, 0.01 * c_hi)

            d = bb[p, pl.ds(64, 16)]
            for i in range(32):
                gs = bcast(c_lo, c_hi, i)
                d = d + gs * w2b[p, pl.ds(i * C1, 16)]

            m = jnp.max(d)
            ind3 = plsc.all_reduce_ffs(d == m)
            return jnp.where(lanes == p, ind3, res)

        ind3v = lax.fori_loop(0, _GP, pix, jnp.full((16,), 0, _I32))
        i12v = i12_all[pl.ds(g * _GP, _GP)]
        res = jnp.clip(i12v * C3 + (ind3v - PAD3), 0, C12 * C3 - 1)
        plsc.store_scatter(res_all, [g * _GP + lanes], res)

    issue(0, 0)
    issue(1, 1)

    def body(gp, carry):
        for b in (0, 1):
            g = gp * 2 + b
            drain(g, b)
            compute(g, b)

            @pl.when(g + 2 < _NG)
            def _():
                issue(g + 2, b)
        return carry

    lax.fori_loop(0, _NG // 2, body, 0)
    pltpu.sync_copy(res_all, outh.at[pl.ds(wid * _PW, _PW)])


def _stage3(xc, i12l, i12, w30r, w31r, w32r, bcat):
    fn = pl.kernel(
        _stage3_body,
        out_type=jax.ShapeDtypeStruct((NPIX,), _I32),
        mesh=_sc_mesh(),
        compiler_params=pltpu.CompilerParams(needs_layout_passes=False),
        scratch_types=[
            pltpu.VMEM((_PW,), _I32),
            pltpu.VMEM((_PW,), _I32),
            pltpu.VMEM((_PW,), _I32),
            pltpu.VMEM((32, _GP), _F32),
            pltpu.VMEM((32, _GP), _F32),
            pltpu.VMEM((_GP, 32), _F32),
            pltpu.VMEM((_GP, 32), _F32),
            pltpu.VMEM((_GP, 1024), _F32),
            pltpu.VMEM((_GP, 1024), _F32),
            pltpu.VMEM((_GP, 1024), _F32),
            pltpu.VMEM((_GP, 1024), _F32),
            pltpu.VMEM((_GP, 512), _F32),
            pltpu.VMEM((_GP, 512), _F32),
            pltpu.VMEM((_GP, 128), _F32),
            pltpu.VMEM((_GP, 128), _F32),
            pltpu.SemaphoreType.DMA,
            pltpu.SemaphoreType.DMA,
        ],
    )
    return fn(xc, i12l, i12, w30r, w31r, w32r, bcat)


def kernel(x_in, w1_0, b1_0, w1_1, b1_1, w1_2, b1_2,
           w2_0, b2_0, w2_1, b2_1, w2_2, b2_2,
           w3_0, b3_0, w3_1, b3_1, w3_2, b3_2):
    w2t0 = w2_0.transpose(0, 2, 1)
    w2t1 = w2_1.transpose(0, 2, 1)
    w2t2 = w2_2.transpose(0, 2, 1)

    xr = x_in[0].transpose(1, 0, 2)
    i12_3d, i12l_3d = _stage12(
        xr, w1_0, b1_0[:, :, None], w1_1, b1_1[:, :, None],
        w1_2, b1_2[:, :, None],
        w2t0, b2_0[:, :, None], w2t1, b2_1[:, :, None],
        w2t2, b2_2[:, :, None])
    i12 = i12_3d.reshape(NPIX)
    i12l = i12l_3d.reshape(NPIX)

    xc = x_in[0, 64:96].reshape(32, NPIX).T
    w30r = w3_0.reshape(NUM3, 32 * 32)
    w31r = w3_1.reshape(NUM3, 32 * 32)
    w32r = w3_2.reshape(NUM3, 32 * C1)
    bcat = jnp.concatenate(
        [b3_0, b3_1, b3_2, jnp.zeros((NUM3, 128 - 64 - C1), _F32)], axis=1)
    out = _stage3(xc, i12l, i12, w30r, w31r, w32r, bcat)
    return out.reshape(1, 1, H, W)


# speedup vs baseline: 41.9127x; 1.0619x over previous
import functools

import jax
import jax.numpy as jnp
from jax import lax
from jax.experimental import pallas as pl
from jax.experimental.pallas import tpu as pltpu
from jax.experimental.pallas import tpu_sc as plsc

H = 224
W = 256
C1, C2, C3 = 16, 12, 8
PAD2, PAD3 = 2, 4
C12 = C1 * C2
NPIX = H * W
NUM2 = H * C1
NUM3 = H * C12

_F32 = jnp.float32
_I32 = jnp.int32


def _lrelu(x):
    return jnp.where(x >= 0, x, 0.01 * x)


def _mm(a, b):
    return lax.dot_general(
        a, b, (((1,), (0,)), ((), ())),
        preferred_element_type=_F32, precision=lax.Precision.DEFAULT)


def _first_argmax_rows(a):
    m = jnp.max(a, axis=0, keepdims=True)
    io = lax.broadcasted_iota(_I32, a.shape, 0)
    return jnp.min(jnp.where(a == m, io, jnp.int32(10 ** 6)), axis=0,
                   keepdims=True)


_LB = 2


def _stage12_body(x_ref, w10, b10, w11, b11, w12, b12,
                  w20, b20, w21, b21, w22, b22, i12_ref, i12l_ref):
    hb = pl.program_id(0)
    for li in range(_LB):
        xa = x_ref[li, 0:32, :]
        xb = x_ref[li, 32:64, :]

        a = _lrelu(_mm(w10[li], xa) + b10[li])
        a = _lrelu(_mm(w11[li], a) + b11[li])
        a = _mm(w12[li], a) + b12[li]
        ind1 = _first_argmax_rows(a)

        def cond_layer(x, w_ref, b_ref, cout, ind1=ind1, li=li):
            wr = w_ref[li * 16:(li + 1) * 16].reshape(16 * cout, 32)
            y = _mm(wr, x) + b_ref[li * 16:(li + 1) * 16].reshape(
                16 * cout, 1)
            sel = jnp.zeros((cout, W), _F32)
            for e in range(16):
                sel = jnp.where(ind1 == e, y[e * cout:(e + 1) * cout, :],
                                sel)
            return sel

        t = _lrelu(cond_layer(xb, w20, b20, 32))
        t = _lrelu(cond_layer(t, w21, b21, 32))
        t = cond_layer(t, w22, b22, C1)
        ind2 = _first_argmax_rows(t)

        i12 = ind1 * C2 + (ind2 - PAD2)
        i12l = jnp.clip(i12, 0, C12 - 1) + C12 * (hb * _LB + li)
        i12_ref[li] = i12
        i12l_ref[li] = i12l


def _stage12(xr, w1_0, b1_0, w1_1, b1_1, w1_2, b1_2,
             w2t0, b2r0, w2t1, b2r1, w2t2, b2r2):
    grid = (H // _LB,)
    line3 = lambda shape: pl.BlockSpec(shape, lambda h: (h, 0, 0))
    out_sd = jax.ShapeDtypeStruct((H, 1, W), _I32)
    return pl.pallas_call(
        _stage12_body,
        grid=grid,
        in_specs=[
            pl.BlockSpec((_LB, 96, W), lambda h: (h, 0, 0)),
            line3((_LB, 32, 32)), line3((_LB, 32, 1)),
            line3((_LB, 32, 32)), line3((_LB, 32, 1)),
            line3((_LB, 16, 32)), line3((_LB, 16, 1)),
            line3((16 * _LB, 32, 32)), line3((16 * _LB, 32, 1)),
            line3((16 * _LB, 32, 32)), line3((16 * _LB, 32, 1)),
            line3((16 * _LB, 16, 32)), line3((16 * _LB, 16, 1)),
        ],
        out_specs=[line3((_LB, 1, W)), line3((_LB, 1, W))],
        out_shape=[out_sd, out_sd],
    )(xr, w1_0, b1_0, w1_1, b1_1, w1_2, b1_2,
      w2t0, b2r0, w2t1, b2r1, w2t2, b2r2)


_NW = 32
_PW = NPIX // _NW
_GP = 16
_NG = _PW // _GP


def _sc_mesh():
    return plsc.VectorSubcoreMesh(core_axis_name="c", subcore_axis_name="s",
                                  num_cores=2, num_subcores=16)


def _stage3_body(xc, idxh, i12h, w30, w31, w32, bcat, outh,
                 idx_all, i12_all, res_all, h1buf, h2buf,
                 xb_0, xb_1, w0_0, w0_1, w1_0b, w1_1b, w2_0b, w2_1b,
                 bb_0, bb_1, sem0, sem1):
    cid = lax.axis_index("c")
    sid = lax.axis_index("s")
    wid = sid * 2 + cid
    lanes = lax.iota(_I32, 16)
    xbufs, w0bufs, w1bufs = (xb_0, xb_1), (w0_0, w0_1), (w1_0b, w1_1b)
    w2bufs, bbufs, sems = (w2_0b, w2_1b), (bb_0, bb_1), (sem0, sem1)

    pltpu.sync_copy(idxh.at[pl.ds(wid * _PW, _PW)], idx_all)
    pltpu.sync_copy(i12h.at[pl.ds(wid * _PW, _PW)], i12_all)

    def copies(g, b):
        idxs = idx_all.at[pl.ds(g * _GP, _GP)]
        return (
            (xc.at[pl.ds(wid * _PW + g * _GP, _GP)], xbufs[b]),
            (w30.at[idxs], w0bufs[b]),
            (w31.at[idxs], w1bufs[b]),
            (w32.at[idxs], w2bufs[b]),
            (bcat.at[idxs], bbufs[b]),
        )

    def issue(g, b):
        for src, dst in copies(g, b):
            pltpu.async_copy(src, dst, sems[b])

    def drain(g, b):
        for src, dst in copies(g, b):
            pltpu.make_async_copy(src, dst, sems[b]).wait()

    def compute(g, b):
        xbuf, w0b, w1b = xbufs[b], w0bufs[b], w1bufs[b]
        w2b, bb = w2bufs[b], bbufs[b]

        def bcast(lo, hi, i):
            src = lo if i < 16 else hi
            idx = jnp.full((16, 1), i % 16, _I32)
            return lax.gather(
                src, idx,
                lax.GatherDimensionNumbers(
                    offset_dims=(), collapsed_slice_dims=(0,),
                    start_index_map=(0,)),
                (1,), mode=lax.GatherScatterMode.PROMISE_IN_BOUNDS)

        def pix(p, res):
            x_lo = xbuf[p, pl.ds(0, 16)]
            x_hi = xbuf[p, pl.ds(16, 16)]
            a_lo = bb[p, pl.ds(0, 16)]
            a_hi = bb[p, pl.ds(16, 16)]
            for i in range(32):
                xs = bcast(x_lo, x_hi, i)
                a_lo = a_lo + xs * w0b[p, pl.ds(i * 32, 16)]
                a_hi = a_hi + xs * w0b[p, pl.ds(i * 32 + 16, 16)]
            a_lo = jnp.where(a_lo >= 0, a_lo, 0.01 * a_lo)
            a_hi = jnp.where(a_hi >= 0, a_hi, 0.01 * a_hi)

            c_lo = bb[p, pl.ds(32, 16)]
            c_hi = bb[p, pl.ds(48, 16)]
            for i in range(32):
                hs = bcast(a_lo, a_hi, i)
                c_lo = c_lo + hs * w1b[p, pl.ds(i * 32, 16)]
                c_hi = c_hi + hs * w1b[p, pl.ds(i * 32 + 16, 16)]
            c_lo = jnp.where(c_lo >= 0, c_lo, 0.01 * c_lo)
            c_hi = jnp.where(c_hi >= 0, c_hi, 0.01 * c_hi)

            d = bb[p, pl.ds(64, 16)]
            for i in range(32):
                gs = bcast(c_lo, c_hi, i)
                d = d + gs * w2b[p, pl.ds(i * C1, 16)]

            m = jnp.max(d)
            ind3 = plsc.all_reduce_ffs(d == m)
            return jnp.where(lanes == p, ind3, res)

        ind3v = lax.fori_loop(0, _GP, pix, jnp.full((16,), 0, _I32))
        i12v = i12_all[pl.ds(g * _GP, _GP)]
        res = jnp.clip(i12v * C3 + (ind3v - PAD3), 0, C12 * C3 - 1)
        plsc.store_scatter(res_all, [g * _GP + lanes], res)

    issue(0, 0)
    issue(1, 1)

    def body(gp, carry):
        for b in (0, 1):
            g = gp * 2 + b
            drain(g, b)
            compute(g, b)

            @pl.when(g + 2 < _NG)
            def _():
                issue(g + 2, b)
        return carry

    lax.fori_loop(0, _NG // 2, body, 0)
    pltpu.sync_copy(res_all, outh.at[pl.ds(wid * _PW, _PW)])


def _stage3(xc, i12l, i12, w30r, w31r, w32r, bcat):
    fn = pl.kernel(
        _stage3_body,
        out_type=jax.ShapeDtypeStruct((NPIX,), _I32),
        mesh=_sc_mesh(),
        compiler_params=pltpu.CompilerParams(needs_layout_passes=False),
        scratch_types=[
            pltpu.VMEM((_PW,), _I32),
            pltpu.VMEM((_PW,), _I32),
            pltpu.VMEM((_PW,), _I32),
            pltpu.VMEM((32, _GP), _F32),
            pltpu.VMEM((32, _GP), _F32),
            pltpu.VMEM((_GP, 32), _F32),
            pltpu.VMEM((_GP, 32), _F32),
            pltpu.VMEM((_GP, 1024), _F32),
            pltpu.VMEM((_GP, 1024), _F32),
            pltpu.VMEM((_GP, 1024), _F32),
            pltpu.VMEM((_GP, 1024), _F32),
            pltpu.VMEM((_GP, 512), _F32),
            pltpu.VMEM((_GP, 512), _F32),
            pltpu.VMEM((_GP, 128), _F32),
            pltpu.VMEM((_GP, 128), _F32),
            pltpu.SemaphoreType.DMA,
            pltpu.SemaphoreType.DMA,
        ],
    )
    return fn(xc, i12l, i12, w30r, w31r, w32r, bcat)


def kernel(x_in, w1_0, b1_0, w1_1, b1_1, w1_2, b1_2,
           w2_0, b2_0, w2_1, b2_1, w2_2, b2_2,
           w3_0, b3_0, w3_1, b3_1, w3_2, b3_2):
    w2t0 = w2_0.transpose(0, 2, 1)
    w2t1 = w2_1.transpose(0, 2, 1)
    w2t2 = w2_2.transpose(0, 2, 1)

    xr = x_in[0].transpose(1, 0, 2)
    i12_3d, i12l_3d = _stage12(
        xr, w1_0, b1_0[:, :, None], w1_1, b1_1[:, :, None],
        w1_2, b1_2[:, :, None],
        w2t0, b2_0[:, :, None], w2t1, b2_1[:, :, None],
        w2t2, b2_2[:, :, None])
    i12 = i12_3d.reshape(NPIX)
    i12l = i12l_3d.reshape(NPIX)

    xc = x_in[0, 64:96].reshape(32, NPIX).T
    w30r = w3_0.reshape(NUM3, 32 * 32)
    w31r = w3_1.reshape(NUM3, 32 * 32)
    w32r = w3_2.reshape(NUM3, 32 * C1)
    bcat = jnp.concatenate(
        [b3_0, b3_1, b3_2, jnp.zeros((NUM3, 128 - 64 - C1), _F32)], axis=1)
    out = _stage3(xc, i12l, i12, w30r, w31r, w32r, bcat)
    return out.reshape(1, 1, H, W)


# speedup vs baseline: 43.0143x; 1.0263x over previous
import functools

import jax
import jax.numpy as jnp
from jax import lax
from jax.experimental import pallas as pl
from jax.experimental.pallas import tpu as pltpu
from jax.experimental.pallas import tpu_sc as plsc

H = 224
W = 256
C1, C2, C3 = 16, 12, 8
PAD2, PAD3 = 2, 4
C12 = C1 * C2
NPIX = H * W
NUM2 = H * C1
NUM3 = H * C12

_F32 = jnp.float32
_I32 = jnp.int32


def _lrelu(x):
    return jnp.where(x >= 0, x, 0.01 * x)


def _mm(a, b):
    return lax.dot_general(
        a, b, (((1,), (0,)), ((), ())),
        preferred_element_type=_F32, precision=lax.Precision.DEFAULT)


def _first_argmax_rows(a):
    m = jnp.max(a, axis=0, keepdims=True)
    io = lax.broadcasted_iota(_I32, a.shape, 0)
    return jnp.min(jnp.where(a == m, io, jnp.int32(10 ** 6)), axis=0,
                   keepdims=True)


_LB = 4


def _stage12_body(x_ref, w10, b10, w11, b11, w12, b12,
                  w20, b20, w21, b21, w22, b22, i12_ref, i12l_ref):
    hb = pl.program_id(0)
    for li in range(_LB):
        xa = x_ref[li, 0:32, :]
        xb = x_ref[li, 32:64, :]

        a = _lrelu(_mm(w10[li], xa) + b10[li])
        a = _lrelu(_mm(w11[li], a) + b11[li])
        a = _mm(w12[li], a) + b12[li]
        ind1 = _first_argmax_rows(a)

        def cond_layer(x, w_ref, b_ref, cout, ind1=ind1, li=li):
            wr = w_ref[li * 16:(li + 1) * 16].reshape(16 * cout, 32)
            y = _mm(wr, x) + b_ref[li * 16:(li + 1) * 16].reshape(
                16 * cout, 1)
            sel = jnp.zeros((cout, W), _F32)
            for e in range(16):
                sel = jnp.where(ind1 == e, y[e * cout:(e + 1) * cout, :],
                                sel)
            return sel

        t = _lrelu(cond_layer(xb, w20, b20, 32))
        t = _lrelu(cond_layer(t, w21, b21, 32))
        t = cond_layer(t, w22, b22, C1)
        ind2 = _first_argmax_rows(t)

        i12 = ind1 * C2 + (ind2 - PAD2)
        i12l = jnp.clip(i12, 0, C12 - 1) + C12 * (hb * _LB + li)
        i12_ref[li] = i12
        i12l_ref[li] = i12l


def _stage12(xr, w1_0, b1_0, w1_1, b1_1, w1_2, b1_2,
             w2t0, b2r0, w2t1, b2r1, w2t2, b2r2):
    grid = (H // _LB,)
    line3 = lambda shape: pl.BlockSpec(shape, lambda h: (h, 0, 0))
    out_sd = jax.ShapeDtypeStruct((H, 1, W), _I32)
    return pl.pallas_call(
        _stage12_body,
        grid=grid,
        in_specs=[
            pl.BlockSpec((_LB, 96, W), lambda h: (h, 0, 0)),
            line3((_LB, 32, 32)), line3((_LB, 32, 1)),
            line3((_LB, 32, 32)), line3((_LB, 32, 1)),
            line3((_LB, 16, 32)), line3((_LB, 16, 1)),
            line3((16 * _LB, 32, 32)), line3((16 * _LB, 32, 1)),
            line3((16 * _LB, 32, 32)), line3((16 * _LB, 32, 1)),
            line3((16 * _LB, 16, 32)), line3((16 * _LB, 16, 1)),
        ],
        out_specs=[line3((_LB, 1, W)), line3((_LB, 1, W))],
        out_shape=[out_sd, out_sd],
    )(xr, w1_0, b1_0, w1_1, b1_1, w1_2, b1_2,
      w2t0, b2r0, w2t1, b2r1, w2t2, b2r2)


_NW = 32
_PW = NPIX // _NW
_GP = 16
_NG = _PW // _GP


def _sc_mesh():
    return plsc.VectorSubcoreMesh(core_axis_name="c", subcore_axis_name="s",
                                  num_cores=2, num_subcores=16)


def _stage3_body(xc, idxh, i12h, w30, w31, w32, bcat, outh,
                 idx_all, i12_all, res_all, h1buf, h2buf,
                 xb_0, xb_1, w0_0, w0_1, w1_0b, w1_1b, w2_0b, w2_1b,
                 bb_0, bb_1, sem0, sem1):
    cid = lax.axis_index("c")
    sid = lax.axis_index("s")
    wid = sid * 2 + cid
    lanes = lax.iota(_I32, 16)
    xbufs, w0bufs, w1bufs = (xb_0, xb_1), (w0_0, w0_1), (w1_0b, w1_1b)
    w2bufs, bbufs, sems = (w2_0b, w2_1b), (bb_0, bb_1), (sem0, sem1)

    pltpu.sync_copy(idxh.at[pl.ds(wid * _PW, _PW)], idx_all)
    pltpu.sync_copy(i12h.at[pl.ds(wid * _PW, _PW)], i12_all)

    def copies(g, b):
        idxs = idx_all.at[pl.ds(g * _GP, _GP)]
        return (
            (xc.at[pl.ds(wid * _PW + g * _GP, _GP)], xbufs[b]),
            (w30.at[idxs], w0bufs[b]),
            (w31.at[idxs], w1bufs[b]),
            (w32.at[idxs], w2bufs[b]),
            (bcat.at[idxs], bbufs[b]),
        )

    def issue(g, b):
        for src, dst in copies(g, b):
            pltpu.async_copy(src, dst, sems[b])

    def drain(g, b):
        for src, dst in copies(g, b):
            pltpu.make_async_copy(src, dst, sems[b]).wait()

    def compute(g, b):
        xbuf, w0b, w1b = xbufs[b], w0bufs[b], w1bufs[b]
        w2b, bb = w2bufs[b], bbufs[b]

        def bcast(lo, hi, i):
            src = lo if i < 16 else hi
            idx = jnp.full((16, 1), i % 16, _I32)
            return lax.gather(
                src, idx,
                lax.GatherDimensionNumbers(
                    offset_dims=(), collapsed_slice_dims=(0,),
                    start_index_map=(0,)),
                (1,), mode=lax.GatherScatterMode.PROMISE_IN_BOUNDS)

        def pix(p, res):
            x_lo = xbuf[p, pl.ds(0, 16)]
            x_hi = xbuf[p, pl.ds(16, 16)]
            a_lo = bb[p, pl.ds(0, 16)]
            a_hi = bb[p, pl.ds(16, 16)]
            for i in range(32):
                xs = bcast(x_lo, x_hi, i)
                a_lo = a_lo + xs * w0b[p, pl.ds(i * 32, 16)]
                a_hi = a_hi + xs * w0b[p, pl.ds(i * 32 + 16, 16)]
            a_lo = jnp.where(a_lo >= 0, a_lo, 0.01 * a_lo)
            a_hi = jnp.where(a_hi >= 0, a_hi, 0.01 * a_hi)

            c_lo = bb[p, pl.ds(32, 16)]
            c_hi = bb[p, pl.ds(48, 16)]
            for i in range(32):
                hs = bcast(a_lo, a_hi, i)
                c_lo = c_lo + hs * w1b[p, pl.ds(i * 32, 16)]
                c_hi = c_hi + hs * w1b[p, pl.ds(i * 32 + 16, 16)]
            c_lo = jnp.where(c_lo >= 0, c_lo, 0.01 * c_lo)
            c_hi = jnp.where(c_hi >= 0, c_hi, 0.01 * c_hi)

            d = bb[p, pl.ds(64, 16)]
            for i in range(32):
                gs = bcast(c_lo, c_hi, i)
                d = d + gs * w2b[p, pl.ds(i * C1, 16)]

            m = jnp.max(d)
            ind3 = plsc.all_reduce_ffs(d == m)
            return jnp.where(lanes == p, ind3, res)

        ind3v = lax.fori_loop(0, _GP, pix, jnp.full((16,), 0, _I32))
        i12v = i12_all[pl.ds(g * _GP, _GP)]
        res = jnp.clip(i12v * C3 + (ind3v - PAD3), 0, C12 * C3 - 1)
        plsc.store_scatter(res_all, [g * _GP + lanes], res)

    issue(0, 0)
    issue(1, 1)

    def body(gp, carry):
        for b in (0, 1):
            g = gp * 2 + b
            drain(g, b)
            compute(g, b)

            @pl.when(g + 2 < _NG)
            def _():
                issue(g + 2, b)
        return carry

    lax.fori_loop(0, _NG // 2, body, 0)
    pltpu.sync_copy(res_all, outh.at[pl.ds(wid * _PW, _PW)])


def _stage3(xc, i12l, i12, w30r, w31r, w32r, bcat):
    fn = pl.kernel(
        _stage3_body,
        out_type=jax.ShapeDtypeStruct((NPIX,), _I32),
        mesh=_sc_mesh(),
        compiler_params=pltpu.CompilerParams(needs_layout_passes=False),
        scratch_types=[
            pltpu.VMEM((_PW,), _I32),
            pltpu.VMEM((_PW,), _I32),
            pltpu.VMEM((_PW,), _I32),
            pltpu.VMEM((32, _GP), _F32),
            pltpu.VMEM((32, _GP), _F32),
            pltpu.VMEM((_GP, 32), _F32),
            pltpu.VMEM((_GP, 32), _F32),
            pltpu.VMEM((_GP, 1024), _F32),
            pltpu.VMEM((_GP, 1024), _F32),
            pltpu.VMEM((_GP, 1024), _F32),
            pltpu.VMEM((_GP, 1024), _F32),
            pltpu.VMEM((_GP, 512), _F32),
            pltpu.VMEM((_GP, 512), _F32),
            pltpu.VMEM((_GP, 128), _F32),
            pltpu.VMEM((_GP, 128), _F32),
            pltpu.SemaphoreType.DMA,
            pltpu.SemaphoreType.DMA,
        ],
    )
    return fn(xc, i12l, i12, w30r, w31r, w32r, bcat)


def kernel(x_in, w1_0, b1_0, w1_1, b1_1, w1_2, b1_2,
           w2_0, b2_0, w2_1, b2_1, w2_2, b2_2,
           w3_0, b3_0, w3_1, b3_1, w3_2, b3_2):
    w2t0 = w2_0.transpose(0, 2, 1)
    w2t1 = w2_1.transpose(0, 2, 1)
    w2t2 = w2_2.transpose(0, 2, 1)

    xr = x_in[0].transpose(1, 0, 2)
    i12_3d, i12l_3d = _stage12(
        xr, w1_0, b1_0[:, :, None], w1_1, b1_1[:, :, None],
        w1_2, b1_2[:, :, None],
        w2t0, b2_0[:, :, None], w2t1, b2_1[:, :, None],
        w2t2, b2_2[:, :, None])
    i12 = i12_3d.reshape(NPIX)
    i12l = i12l_3d.reshape(NPIX)

    xc = x_in[0, 64:96].reshape(32, NPIX).T
    w30r = w3_0.reshape(NUM3, 32 * 32)
    w31r = w3_1.reshape(NUM3, 32 * 32)
    w32r = w3_2.reshape(NUM3, 32 * C1)
    bcat = jnp.concatenate(
        [b3_0, b3_1, b3_2, jnp.zeros((NUM3, 128 - 64 - C1), _F32)], axis=1)
    out = _stage3(xc, i12l, i12, w30r, w31r, w32r, bcat)
    return out.reshape(1, 1, H, W)


# speedup vs baseline: 43.3271x; 1.0073x over previous
import functools

import jax
import jax.numpy as jnp
from jax import lax
from jax.experimental import pallas as pl
from jax.experimental.pallas import tpu as pltpu
from jax.experimental.pallas import tpu_sc as plsc

H = 224
W = 256
C1, C2, C3 = 16, 12, 8
PAD2, PAD3 = 2, 4
C12 = C1 * C2
NPIX = H * W
NUM2 = H * C1
NUM3 = H * C12

_F32 = jnp.float32
_I32 = jnp.int32


def _lrelu(x):
    return jnp.where(x >= 0, x, 0.01 * x)


def _mm(a, b):
    return lax.dot_general(
        a, b, (((1,), (0,)), ((), ())),
        preferred_element_type=_F32, precision=lax.Precision.DEFAULT)


def _first_argmax_rows(a):
    m = jnp.max(a, axis=0, keepdims=True)
    io = lax.broadcasted_iota(_I32, a.shape, 0)
    return jnp.min(jnp.where(a == m, io, jnp.int32(10 ** 6)), axis=0,
                   keepdims=True)


_LB = 8


def _stage12_body(x_ref, w10, b10, w11, b11, w12, b12,
                  w20, b20, w21, b21, w22, b22, i12_ref, i12l_ref):
    hb = pl.program_id(0)
    for li in range(_LB):
        xa = x_ref[li, 0:32, :]
        xb = x_ref[li, 32:64, :]

        a = _lrelu(_mm(w10[li], xa) + b10[li])
        a = _lrelu(_mm(w11[li], a) + b11[li])
        a = _mm(w12[li], a) + b12[li]
        ind1 = _first_argmax_rows(a)

        def cond_layer(x, w_ref, b_ref, cout, ind1=ind1, li=li):
            wr = w_ref[li * 16:(li + 1) * 16].reshape(16 * cout, 32)
            y = _mm(wr, x) + b_ref[li * 16:(li + 1) * 16].reshape(
                16 * cout, 1)
            sel = jnp.zeros((cout, W), _F32)
            for e in range(16):
                sel = jnp.where(ind1 == e, y[e * cout:(e + 1) * cout, :],
                                sel)
            return sel

        t = _lrelu(cond_layer(xb, w20, b20, 32))
        t = _lrelu(cond_layer(t, w21, b21, 32))
        t = cond_layer(t, w22, b22, C1)
        ind2 = _first_argmax_rows(t)

        i12 = ind1 * C2 + (ind2 - PAD2)
        i12l = jnp.clip(i12, 0, C12 - 1) + C12 * (hb * _LB + li)
        i12_ref[li] = i12
        i12l_ref[li] = i12l


def _stage12(xr, w1_0, b1_0, w1_1, b1_1, w1_2, b1_2,
             w2t0, b2r0, w2t1, b2r1, w2t2, b2r2):
    grid = (H // _LB,)
    line3 = lambda shape: pl.BlockSpec(shape, lambda h: (h, 0, 0))
    out_sd = jax.ShapeDtypeStruct((H, 1, W), _I32)
    return pl.pallas_call(
        _stage12_body,
        grid=grid,
        in_specs=[
            pl.BlockSpec((_LB, 96, W), lambda h: (h, 0, 0)),
            line3((_LB, 32, 32)), line3((_LB, 32, 1)),
            line3((_LB, 32, 32)), line3((_LB, 32, 1)),
            line3((_LB, 16, 32)), line3((_LB, 16, 1)),
            line3((16 * _LB, 32, 32)), line3((16 * _LB, 32, 1)),
            line3((16 * _LB, 32, 32)), line3((16 * _LB, 32, 1)),
            line3((16 * _LB, 16, 32)), line3((16 * _LB, 16, 1)),
        ],
        out_specs=[line3((_LB, 1, W)), line3((_LB, 1, W))],
        out_shape=[out_sd, out_sd],
    )(xr, w1_0, b1_0, w1_1, b1_1, w1_2, b1_2,
      w2t0, b2r0, w2t1, b2r1, w2t2, b2r2)


_NW = 32
_PW = NPIX // _NW
_GP = 16
_NG = _PW // _GP


def _sc_mesh():
    return plsc.VectorSubcoreMesh(core_axis_name="c", subcore_axis_name="s",
                                  num_cores=2, num_subcores=16)


def _stage3_body(xc, idxh, i12h, w30, w31, w32, bcat, outh,
                 idx_all, i12_all, res_all, h1buf, h2buf,
                 xb_0, xb_1, w0_0, w0_1, w1_0b, w1_1b, w2_0b, w2_1b,
                 bb_0, bb_1, sem0, sem1):
    cid = lax.axis_index("c")
    sid = lax.axis_index("s")
    wid = sid * 2 + cid
    lanes = lax.iota(_I32, 16)
    xbufs, w0bufs, w1bufs = (xb_0, xb_1), (w0_0, w0_1), (w1_0b, w1_1b)
    w2bufs, bbufs, sems = (w2_0b, w2_1b), (bb_0, bb_1), (sem0, sem1)

    pltpu.sync_copy(idxh.at[pl.ds(wid * _PW, _PW)], idx_all)
    pltpu.sync_copy(i12h.at[pl.ds(wid * _PW, _PW)], i12_all)

    def copies(g, b):
        idxs = idx_all.at[pl.ds(g * _GP, _GP)]
        return (
            (xc.at[pl.ds(wid * _PW + g * _GP, _GP)], xbufs[b]),
            (w30.at[idxs], w0bufs[b]),
            (w31.at[idxs], w1bufs[b]),
            (w32.at[idxs], w2bufs[b]),
            (bcat.at[idxs], bbufs[b]),
        )

    def issue(g, b):
        for src, dst in copies(g, b):
            pltpu.async_copy(src, dst, sems[b])

    def drain(g, b):
        for src, dst in copies(g, b):
            pltpu.make_async_copy(src, dst, sems[b]).wait()

    def compute(g, b):
        xbuf, w0b, w1b = xbufs[b], w0bufs[b], w1bufs[b]
        w2b, bb = w2bufs[b], bbufs[b]

        def bcast(lo, hi, i):
            src = lo if i < 16 else hi
            idx = jnp.full((16, 1), i % 16, _I32)
            return lax.gather(
                src, idx,
                lax.GatherDimensionNumbers(
                    offset_dims=(), collapsed_slice_dims=(0,),
                    start_index_map=(0,)),
                (1,), mode=lax.GatherScatterMode.PROMISE_IN_BOUNDS)

        def pix(p, res):
            x_lo = xbuf[p, pl.ds(0, 16)]
            x_hi = xbuf[p, pl.ds(16, 16)]
            a_lo = bb[p, pl.ds(0, 16)]
            a_hi = bb[p, pl.ds(16, 16)]
            for i in range(32):
                xs = bcast(x_lo, x_hi, i)
                a_lo = a_lo + xs * w0b[p, pl.ds(i * 32, 16)]
                a_hi = a_hi + xs * w0b[p, pl.ds(i * 32 + 16, 16)]
            a_lo = jnp.where(a_lo >= 0, a_lo, 0.01 * a_lo)
            a_hi = jnp.where(a_hi >= 0, a_hi, 0.01 * a_hi)

            c_lo = bb[p, pl.ds(32, 16)]
            c_hi = bb[p, pl.ds(48, 16)]
            for i in range(32):
                hs = bcast(a_lo, a_hi, i)
                c_lo = c_lo + hs * w1b[p, pl.ds(i * 32, 16)]
                c_hi = c_hi + hs * w1b[p, pl.ds(i * 32 + 16, 16)]
            c_lo = jnp.where(c_lo >= 0, c_lo, 0.01 * c_lo)
            c_hi = jnp.where(c_hi >= 0, c_hi, 0.01 * c_hi)

            d = bb[p, pl.ds(64, 16)]
            for i in range(32):
                gs = bcast(c_lo, c_hi, i)
                d = d + gs * w2b[p, pl.ds(i * C1, 16)]

            m = jnp.max(d)
            ind3 = plsc.all_reduce_ffs(d == m)
            return jnp.where(lanes == p, ind3, res)

        ind3v = lax.fori_loop(0, _GP, pix, jnp.full((16,), 0, _I32))
        i12v = i12_all[pl.ds(g * _GP, _GP)]
        res = jnp.clip(i12v * C3 + (ind3v - PAD3), 0, C12 * C3 - 1)
        plsc.store_scatter(res_all, [g * _GP + lanes], res)

    issue(0, 0)
    issue(1, 1)

    def body(gp, carry):
        for b in (0, 1):
            g = gp * 2 + b
            drain(g, b)
            compute(g, b)

            @pl.when(g + 2 < _NG)
            def _():
                issue(g + 2, b)
        return carry

    lax.fori_loop(0, _NG // 2, body, 0)
    pltpu.sync_copy(res_all, outh.at[pl.ds(wid * _PW, _PW)])


def _stage3(xc, i12l, i12, w30r, w31r, w32r, bcat):
    fn = pl.kernel(
        _stage3_body,
        out_type=jax.ShapeDtypeStruct((NPIX,), _I32),
        mesh=_sc_mesh(),
        compiler_params=pltpu.CompilerParams(needs_layout_passes=False),
        scratch_types=[
            pltpu.VMEM((_PW,), _I32),
            pltpu.VMEM((_PW,), _I32),
            pltpu.VMEM((_PW,), _I32),
            pltpu.VMEM((32, _GP), _F32),
            pltpu.VMEM((32, _GP), _F32),
            pltpu.VMEM((_GP, 32), _F32),
            pltpu.VMEM((_GP, 32), _F32),
            pltpu.VMEM((_GP, 1024), _F32),
            pltpu.VMEM((_GP, 1024), _F32),
            pltpu.VMEM((_GP, 1024), _F32),
            pltpu.VMEM((_GP, 1024), _F32),
            pltpu.VMEM((_GP, 512), _F32),
            pltpu.VMEM((_GP, 512), _F32),
            pltpu.VMEM((_GP, 128), _F32),
            pltpu.VMEM((_GP, 128), _F32),
            pltpu.SemaphoreType.DMA,
            pltpu.SemaphoreType.DMA,
        ],
    )
    return fn(xc, i12l, i12, w30r, w31r, w32r, bcat)


def kernel(x_in, w1_0, b1_0, w1_1, b1_1, w1_2, b1_2,
           w2_0, b2_0, w2_1, b2_1, w2_2, b2_2,
           w3_0, b3_0, w3_1, b3_1, w3_2, b3_2):
    w2t0 = w2_0.transpose(0, 2, 1)
    w2t1 = w2_1.transpose(0, 2, 1)
    w2t2 = w2_2.transpose(0, 2, 1)

    xr = x_in[0].transpose(1, 0, 2)
    i12_3d, i12l_3d = _stage12(
        xr, w1_0, b1_0[:, :, None], w1_1, b1_1[:, :, None],
        w1_2, b1_2[:, :, None],
        w2t0, b2_0[:, :, None], w2t1, b2_1[:, :, None],
        w2t2, b2_2[:, :, None])
    i12 = i12_3d.reshape(NPIX)
    i12l = i12l_3d.reshape(NPIX)

    xc = x_in[0, 64:96].reshape(32, NPIX).T
    w30r = w3_0.reshape(NUM3, 32 * 32)
    w31r = w3_1.reshape(NUM3, 32 * 32)
    w32r = w3_2.reshape(NUM3, 32 * C1)
    bcat = jnp.concatenate(
        [b3_0, b3_1, b3_2, jnp.zeros((NUM3, 128 - 64 - C1), _F32)], axis=1)
    out = _stage3(xc, i12l, i12, w30r, w31r, w32r, bcat)
    return out.reshape(1, 1, H, W)
